# baseline (speedup 1.0000x reference)
import jax
import jax.numpy as jnp
from jax import lax
from jax.experimental import pallas as pl
from jax.experimental.pallas import tpu as pltpu
from jax.experimental.pallas import tpu_sc as plsc

VOCAB_N = 100000
EMBED_D = 512
SEQ_N = 50
BATCH_N = 1024

NC = 2
NS = 16
NW = NC * NS

BPW = BATCH_N // NW
SPAD = 56
DEPTH = 3


def _pe_table():
    i = jnp.arange(SEQ_N, dtype=jnp.float32)[:, None]
    j = jnp.arange(EMBED_D // 2, dtype=jnp.float32)[None, :]
    ang = i / jnp.power(jnp.float32(10000.0), 2.0 * j / EMBED_D)
    return jnp.stack([jnp.sin(ang), jnp.cos(ang)], axis=-1).reshape(SEQ_N, EMBED_D)


def _body(table_hbm, x_hbm, pe_hbm, out_hbm,
          idx0, idx1, idx2, buf0, buf1, buf2, pe_v,
          g0, g1, g2, w0, w1, w2):
    wid = lax.axis_index("s") * NC + lax.axis_index("c")
    b0 = wid * BPW

    pltpu.sync_copy(pe_hbm, pe_v)

    idx = (idx0, idx1, idx2)
    buf = (buf0, buf1, buf2)
    gsem = (g0, g1, g2)
    wsem = (w0, w1, w2)

    def fire_gather(k, s):
        pltpu.sync_copy(x_hbm.at[b0 + k], idx[s])
        pltpu.async_copy(table_hbm.at[idx[s].at[pl.ds(0, SPAD)]],
                         buf[s], gsem[s])

    def wait_gather(s):
        pltpu.make_async_copy(table_hbm.at[idx[s].at[pl.ds(0, SPAD)]],
                              buf[s], gsem[s]).wait()

    def wait_writeback(s):
        pltpu.make_async_copy(buf[s], out_hbm.at[b0], wsem[s]).wait()

    def process(k, s):
        wait_gather(s)
        b = buf[s]

        def add_pe(r, _):
            for v in range(0, EMBED_D, 16):
                plsc.addupdate(b.at[r, pl.ds(v, 16)], pe_v[r, pl.ds(v, 16)])
            return 0

        lax.fori_loop(0, SEQ_N, add_pe, 0)
        pltpu.async_copy(b, out_hbm.at[b0 + k], wsem[s])

    def chunk(k, s, fire, wait_wb):
        s2 = (s + 2) % DEPTH
        if wait_wb:
            wait_writeback(s2)
        if fire:
            fire_gather(k + 2, s2)
        process(k, s)

    fire_gather(0, 0)
    fire_gather(1, 1)
    chunk(0, 0, fire=True, wait_wb=False)

    def trip(t, _):
        k = 1 + 3 * t
        chunk(k, 1, fire=True, wait_wb=True)
        chunk(k + 1, 2, fire=True, wait_wb=True)
        chunk(k + 2, 0, fire=True, wait_wb=True)
        return 0

    lax.fori_loop(0, (BPW - 5) // 3, trip, 0)

    chunk(28, 1, fire=True, wait_wb=True)
    chunk(29, 2, fire=True, wait_wb=True)
    chunk(30, 0, fire=False, wait_wb=True)
    chunk(31, 1, fire=False, wait_wb=True)
    wait_writeback(1)


@jax.jit
def _run(x, table, pe):
    x64 = jnp.pad(x, ((0, 0), (0, 64 - SEQ_N)))
    mesh = plsc.VectorSubcoreMesh(core_axis_name="c", subcore_axis_name="s")
    out = pl.kernel(
        _body,
        out_type=jax.ShapeDtypeStruct((BATCH_N, SPAD, EMBED_D), jnp.float32),
        mesh=mesh,
        scratch_types=[
            pltpu.VMEM((64,), jnp.int32),
            pltpu.VMEM((64,), jnp.int32),
            pltpu.VMEM((64,), jnp.int32),
            pltpu.VMEM((SPAD, EMBED_D), jnp.float32),
            pltpu.VMEM((SPAD, EMBED_D), jnp.float32),
            pltpu.VMEM((SPAD, EMBED_D), jnp.float32),
            pltpu.VMEM((SEQ_N, EMBED_D), jnp.float32),
            pltpu.SemaphoreType.DMA,
            pltpu.SemaphoreType.DMA,
            pltpu.SemaphoreType.DMA,
            pltpu.SemaphoreType.DMA,
            pltpu.SemaphoreType.DMA,
            pltpu.SemaphoreType.DMA,
        ],
    )(table, x64, pe)
    return out[:, :SEQ_N, :]


def kernel(x, offsets, table):
    del offsets
    return _run(x, table, _pe_table())


# speedup vs baseline: 1.3626x; 1.3626x over previous
import jax
import jax.numpy as jnp
from jax import lax
from jax.experimental import pallas as pl
from jax.experimental.pallas import tpu as pltpu
from jax.experimental.pallas import tpu_sc as plsc

VOCAB_N = 100000
EMBED_D = 512
SEQ_N = 50
BATCH_N = 1024

NC = 2
NS = 16
NW = NC * NS

ROWS_TOTAL = BATCH_N * SEQ_N
ROWS_PER_W = ROWS_TOTAL // NW
CHUNK = 2 * SEQ_N
CHUNKS_PER_W = ROWS_PER_W // CHUNK


def _pe_table():
    i = jnp.arange(SEQ_N, dtype=jnp.float32)[:, None]
    j = jnp.arange(EMBED_D // 2, dtype=jnp.float32)[None, :]
    ang = i / jnp.power(jnp.float32(10000.0), 2.0 * j / EMBED_D)
    return jnp.stack([jnp.sin(ang), jnp.cos(ang)], axis=-1).reshape(SEQ_N, EMBED_D)


def _body(table_hbm, idx_hbm, pe_hbm, out_hbm,
          idx_v, buf_v, pe_v, gsem, wsem):
    wid = lax.axis_index("s") * NC + lax.axis_index("c")
    j0 = wid * CHUNKS_PER_W

    pltpu.sync_copy(pe_hbm, pe_v)

    for k in range(CHUNKS_PER_W):
        j = j0 + k
        pltpu.sync_copy(idx_hbm.at[j], idx_v)
        pltpu.async_copy(table_hbm.at[idx_v], buf_v, gsem).wait()

        def add_pe(s, _):
            for v in range(0, EMBED_D, 16):
                pev = pe_v[s, pl.ds(v, 16)]
                plsc.addupdate(buf_v.at[s, pl.ds(v, 16)], pev)
                plsc.addupdate(buf_v.at[s + SEQ_N, pl.ds(v, 16)], pev)
            return 0

        lax.fori_loop(0, SEQ_N, add_pe, 0)
        pltpu.async_copy(buf_v, out_hbm.at[j], wsem).wait()


@jax.jit
def _run(x, table, pe):
    idx = x.reshape(ROWS_TOTAL // CHUNK, CHUNK)
    mesh = plsc.VectorSubcoreMesh(core_axis_name="c", subcore_axis_name="s")
    out = pl.kernel(
        _body,
        out_type=jax.ShapeDtypeStruct(
            (ROWS_TOTAL // CHUNK, CHUNK, EMBED_D), jnp.float32),
        mesh=mesh,
        scratch_types=[
            pltpu.VMEM((CHUNK,), jnp.int32),
            pltpu.VMEM((CHUNK, EMBED_D), jnp.float32),
            pltpu.VMEM((SEQ_N, EMBED_D), jnp.float32),
            pltpu.SemaphoreType.DMA,
            pltpu.SemaphoreType.DMA,
        ],
    )(table, idx, pe)
    return out.reshape(BATCH_N, SEQ_N, EMBED_D)


def kernel(x, offsets, table):
    del offsets
    return _run(x, table, _pe_table())
